# baseline (speedup 1.0000x reference)
import functools

import jax
import jax.numpy as jnp
from jax import lax
from jax.experimental import pallas as pl
from jax.experimental.pallas import tpu as pltpu
from jax.experimental.pallas import tpu_sc as plsc

NC = 2
NS = 16
NW = NC * NS

NJ = 26
NI = 16384
D = 32
B = NI * NJ

RPW = B // NW
GR = 1024
NG = RPW // GR

mesh = plsc.VectorSubcoreMesh(core_axis_name="c", subcore_axis_name="s")


@functools.partial(
    pl.kernel,
    mesh=mesh,
    out_type=jax.ShapeDtypeStruct((B, D), jnp.float32),
    scratch_types=[
        pltpu.VMEM((RPW,), jnp.int32),
        pltpu.VMEM((GR, D), jnp.float32),
        pltpu.VMEM((GR, D), jnp.float32),
        pltpu.SemaphoreType.DMA,
        pltpu.SemaphoreType.DMA,
        pltpu.SemaphoreType.DMA,
        pltpu.SemaphoreType.DMA,
    ],
    compiler_params=pltpu.CompilerParams(use_tc_tiling_on_sc=False),
)
def _gather(idxf, wt, inter, ixv, g0, g1, sg0, sg1, sw0, sw1):
    wid = lax.axis_index("s") * NC + lax.axis_index("c")
    base = wid * RPW
    pltpu.sync_copy(idxf.at[pl.ds(base, RPW)], ixv)

    def g_copy(g, gv, sem):
        return pltpu.make_async_copy(
            wt.at[ixv.at[pl.ds(g * GR, GR)]], gv, sem
        )

    def w_copy(g, gv, sem):
        return pltpu.make_async_copy(
            gv, inter.at[pl.ds(base + g * GR, GR)], sem
        )

    g_copy(0, g0, sg0).start()

    def body(g, _):
        @pl.when(lax.rem(g, 2) == 0)
        def _():
            @pl.when(g + 1 < NG)
            def _():
                @pl.when(g >= 1)
                def _():
                    w_copy(g - 1, g1, sw1).wait()

                g_copy(g + 1, g1, sg1).start()

            g_copy(g, g0, sg0).wait()
            w_copy(g, g0, sw0).start()

        @pl.when(lax.rem(g, 2) == 1)
        def _():
            @pl.when(g + 1 < NG)
            def _():
                w_copy(g - 1, g0, sw0).wait()
                g_copy(g + 1, g0, sg0).start()

            g_copy(g, g1, sg1).wait()
            w_copy(g, g1, sw1).start()

        return ()

    lax.fori_loop(0, NG, body, ())
    w_copy(NG - 2, g0, sw0).wait()
    w_copy(NG - 1, g1, sw1).wait()


RCH = 512


def _tc_body(in_ref, out_ref):
    x = in_ref[...]
    parts = [x[:, k4 * D:(k4 + 1) * D].T for k4 in range(4)]
    y = jnp.stack(parts, axis=2).reshape(D, 4 * RCH)
    out_ref[...] = y[None]


_transpose = pl.pallas_call(
    _tc_body,
    out_shape=jax.ShapeDtypeStruct((NJ, D, NI), jnp.float32),
    grid=(NJ, NI // (4 * RCH)),
    in_specs=[
        pl.BlockSpec((RCH, 128), lambda j, t: (j * (NI // (4 * RCH)) + t, 0))
    ],
    out_specs=pl.BlockSpec((1, D, 4 * RCH), lambda j, t: (j, 0, t)),
)


def kernel(idx, weight):
    idxf = idx.T.reshape(-1).astype(jnp.int32)
    inter = _gather(idxf, weight)
    o3 = _transpose(inter.reshape(B // 4, 128))
    return o3.transpose(2, 0, 1).reshape(NI, NJ, D)


# speedup vs baseline: 4.6183x; 4.6183x over previous
import functools

import jax
import jax.numpy as jnp
from jax import lax
from jax.experimental import pallas as pl
from jax.experimental.pallas import tpu as pltpu
from jax.experimental.pallas import tpu_sc as plsc

NC = 2
NS = 16
NW = NC * NS

NJ = 26
NI = 16384
D = 32
B = NI * NJ

RPW = B // NW
GR = 1024
NG = RPW // GR

mesh = plsc.VectorSubcoreMesh(core_axis_name="c", subcore_axis_name="s")


@functools.partial(
    pl.kernel,
    mesh=mesh,
    out_type=jax.ShapeDtypeStruct((B, D), jnp.float32),
    scratch_types=[
        pltpu.VMEM((RPW,), jnp.int32),
        pltpu.VMEM((GR, D), jnp.float32),
        pltpu.VMEM((GR, D), jnp.float32),
        pltpu.SemaphoreType.DMA,
        pltpu.SemaphoreType.DMA,
        pltpu.SemaphoreType.DMA,
        pltpu.SemaphoreType.DMA,
    ],
    compiler_params=pltpu.CompilerParams(use_tc_tiling_on_sc=False),
)
def _gather(idxf, wt, inter, ixv, g0, g1, sg0, sg1, sw0, sw1):
    wid = lax.axis_index("s") * NC + lax.axis_index("c")
    base = wid * RPW
    pltpu.sync_copy(idxf.at[pl.ds(base, RPW)], ixv)

    def g_copy(g, gv, sem):
        return pltpu.make_async_copy(
            wt.at[ixv.at[pl.ds(g * GR, GR)]], gv, sem
        )

    def w_copy(g, gv, sem):
        return pltpu.make_async_copy(
            gv, inter.at[pl.ds(base + g * GR, GR)], sem
        )

    g_copy(0, g0, sg0).start()

    def body(g, _):
        @pl.when(lax.rem(g, 2) == 0)
        def _():
            @pl.when(g + 1 < NG)
            def _():
                @pl.when(g >= 1)
                def _():
                    w_copy(g - 1, g1, sw1).wait()

                g_copy(g + 1, g1, sg1).start()

            g_copy(g, g0, sg0).wait()
            w_copy(g, g0, sw0).start()

        @pl.when(lax.rem(g, 2) == 1)
        def _():
            @pl.when(g + 1 < NG)
            def _():
                w_copy(g - 1, g0, sw0).wait()
                g_copy(g + 1, g0, sg0).start()

            g_copy(g, g1, sg1).wait()
            w_copy(g, g1, sw1).start()

        return ()

    lax.fori_loop(0, NG, body, ())
    w_copy(NG - 2, g0, sw0).wait()
    w_copy(NG - 1, g1, sw1).wait()


RCH = 512
NCH = NI // (4 * RCH)


def _tc_body(in_ref, out_ref):
    x = in_ref[...]
    xt = x.T.reshape(4, D, RCH)
    out_ref[...] = xt.transpose(1, 0, 2).reshape(1, D, 4 * RCH)


_transpose = pl.pallas_call(
    _tc_body,
    out_shape=jax.ShapeDtypeStruct((NJ, D, NI), jnp.float32),
    grid=(NJ, NCH),
    in_specs=[pl.BlockSpec((RCH, 128), lambda j, t: (j * NCH + t, 0))],
    out_specs=pl.BlockSpec((1, D, 4 * RCH), lambda j, t: (j, 0, t)),
)


def kernel(idx, weight):
    idxp = (
        idx.T.reshape(NJ, NCH, 4, RCH)
        .transpose(0, 1, 3, 2)
        .reshape(-1)
        .astype(jnp.int32)
    )
    inter = _gather(idxp, weight)
    o3 = _transpose(inter.reshape(B // 4, 128))
    return o3.transpose(2, 0, 1)
